# baseline (speedup 1.0000x reference)
import functools
import jax
import jax.numpy as jnp
from jax import lax
from jax.experimental import pallas as pl
from jax.experimental.pallas import tpu as pltpu
from jax.experimental.pallas import tpu_sc as plsc

E = 8
NGROUP = 4
GSZ = E // NGROUP
SCALE = 2.5
H = 1024
I = 512
SI = 1024
T = 2048
KTOP = 2

BLK = 128
P_ROWS = T * KTOP + E * BLK
NBLK = P_ROWS // BLK
NA = T * KTOP
NW = 32
A_PER_W = NA // NW
CHUNK = 32


def _silu(v):
    return v / (1.0 + jnp.exp(-v))


def _top1_mask(vals, width):
    m = jnp.max(vals, axis=1, keepdims=True)
    io = lax.broadcasted_iota(jnp.int32, vals.shape, 1)
    idx = jnp.min(jnp.where(vals == m, io, width), axis=1, keepdims=True)
    return io == idx


def _router_body(logits_ref, bias_ref, dest_ref, wvec_ref, bexp_ref):
    logits = logits_ref[...]
    scores = 1.0 / (1.0 + jnp.exp(-logits))
    swb = scores + bias_ref[...]

    e_i = lax.broadcasted_iota(jnp.int32, (E, NGROUP), 0)
    g_i = lax.broadcasted_iota(jnp.int32, (E, NGROUP), 1)
    m_eg = ((e_i // GSZ) == g_i).astype(jnp.float32)
    g_j = lax.broadcasted_iota(jnp.int32, (NGROUP, E), 0)
    e_j = lax.broadcasted_iota(jnp.int32, (NGROUP, E), 1)
    m_ge = ((e_j // GSZ) == g_j).astype(jnp.float32)

    gs = jnp.dot(swb, m_eg, preferred_element_type=jnp.float32,
                 precision=lax.Precision.HIGHEST)
    p1 = _top1_mask(gs, NGROUP)
    p2 = _top1_mask(jnp.where(p1, -1e30, gs), NGROUP)
    gmask = jnp.logical_or(p1, p2).astype(jnp.float32)
    emask = jnp.dot(gmask, m_ge, preferred_element_type=jnp.float32,
                    precision=lax.Precision.HIGHEST)

    masked = jnp.where(emask > 0.5, swb, -1e9)
    oh1 = _top1_mask(masked, E)
    oh2 = _top1_mask(jnp.where(oh1, -1e30, masked), E)
    w1 = jnp.sum(jnp.where(oh1, scores, 0.0), axis=1, keepdims=True)
    w2 = jnp.sum(jnp.where(oh2, scores, 0.0), axis=1, keepdims=True)
    r = SCALE / (w1 + w2 + 1e-20)
    wvec_ref[...] = jnp.concatenate([w1 * r, w2 * r], axis=1)

    ohsum = jnp.where(oh1, 1.0, 0.0) + jnp.where(oh2, 1.0, 0.0)
    c = ohsum
    sh = 1
    while sh < T:
        c = c + jnp.concatenate(
            [jnp.zeros((sh, E), jnp.float32), c[: T - sh, :]], axis=0)
        sh *= 2

    counts = c[T - 1 : T, :]
    ci = counts.astype(jnp.int32)
    pc = ((ci + BLK - 1) // BLK) * BLK
    slt_i = lax.broadcasted_iota(jnp.int32, (E, E), 0)
    slt_j = lax.broadcasted_iota(jnp.int32, (E, E), 1)
    slt = (slt_i < slt_j).astype(jnp.float32)
    off = jnp.dot(pc.astype(jnp.float32), slt,
                  preferred_element_type=jnp.float32,
                  precision=lax.Precision.HIGHEST)

    slot = off + c - 1.0
    d0 = jnp.sum(jnp.where(oh1, slot, 0.0), axis=1, keepdims=True)
    d1 = jnp.sum(jnp.where(oh2, slot, 0.0), axis=1, keepdims=True)
    dest_ref[...] = jnp.concatenate([d0, d1], axis=1).astype(jnp.int32)

    b_i = lax.broadcasted_iota(jnp.int32, (NBLK, E), 0)
    off_b = jnp.broadcast_to(off.astype(jnp.int32), (NBLK, E))
    cmp = (b_i * BLK >= off_b).astype(jnp.int32)
    bexp_ref[...] = jnp.sum(cmp, axis=1, keepdims=True) - 1


def _make_router():
    return pl.pallas_call(
        _router_body,
        out_shape=(
            jax.ShapeDtypeStruct((T, KTOP), jnp.int32),
            jax.ShapeDtypeStruct((T, KTOP), jnp.float32),
            jax.ShapeDtypeStruct((NBLK, 1), jnp.int32),
        ),
        in_specs=[
            pl.BlockSpec((T, E), lambda: (0, 0)),
            pl.BlockSpec((1, E), lambda: (0, 0)),
        ],
        out_specs=(
            pl.BlockSpec((T, KTOP), lambda: (0, 0)),
            pl.BlockSpec((T, KTOP), lambda: (0, 0)),
            pl.BlockSpec((NBLK, 1), lambda: (0, 0)),
        ),
    )


_SC_MESH = plsc.VectorSubcoreMesh(core_axis_name="c", subcore_axis_name="s")


@functools.partial(
    pl.kernel,
    mesh=_SC_MESH,
    out_type=jax.ShapeDtypeStruct((P_ROWS, H), jnp.float32),
    scratch_types=[
        pltpu.VMEM((CHUNK,), jnp.int32),
        pltpu.VMEM((CHUNK,), jnp.int32),
        pltpu.VMEM((CHUNK, H), jnp.float32),
        pltpu.SemaphoreType.DMA,
    ],
)
def _sc_dispatch(x_hbm, dest_hbm, xs_hbm, tok_v, dst_v, rows_v, sem):
    nc = 2
    wid = lax.axis_index("s") * nc + lax.axis_index("c")
    base = wid * A_PER_W
    for s in range(A_PER_W // CHUNK):
        abase = base + s * CHUNK
        for c2 in range(CHUNK // 16):
            v = abase + c2 * 16 + jnp.arange(16, dtype=jnp.int32)
            tok_v[pl.ds(c2 * 16, 16)] = lax.shift_right_logical(v, 1)
        pltpu.sync_copy(dest_hbm.at[pl.ds(abase, CHUNK)], dst_v)
        pltpu.async_copy(x_hbm.at[tok_v], rows_v, sem).wait()
        pltpu.async_copy(rows_v, xs_hbm.at[dst_v], sem).wait()


@functools.partial(
    pl.kernel,
    mesh=_SC_MESH,
    out_type=jax.ShapeDtypeStruct((NA, H), jnp.float32),
    scratch_types=[
        pltpu.VMEM((CHUNK,), jnp.int32),
        pltpu.VMEM((CHUNK, H), jnp.float32),
        pltpu.SemaphoreType.DMA,
    ],
)
def _sc_combine(d_hbm, dest_hbm, g_hbm, dst_v, rows_v, sem):
    nc = 2
    wid = lax.axis_index("s") * nc + lax.axis_index("c")
    base = wid * A_PER_W
    for s in range(A_PER_W // CHUNK):
        abase = base + s * CHUNK
        pltpu.sync_copy(dest_hbm.at[pl.ds(abase, CHUNK)], dst_v)
        pltpu.async_copy(d_hbm.at[dst_v], rows_v, sem).wait()
        pltpu.sync_copy(rows_v, g_hbm.at[pl.ds(abase, CHUNK)])


def _gemm_body(bexp_ref, xs_ref, wg_ref, wu_ref, wd_ref, o_ref):
    xb = xs_ref[...].astype(jnp.bfloat16)
    wg = wg_ref[0].astype(jnp.bfloat16)
    wu = wu_ref[0].astype(jnp.bfloat16)
    wd = wd_ref[0].astype(jnp.bfloat16)
    g = jnp.dot(xb, wg, preferred_element_type=jnp.float32)
    u = jnp.dot(xb, wu, preferred_element_type=jnp.float32)
    a = (_silu(g) * u).astype(jnp.bfloat16)
    o_ref[...] = jnp.dot(a, wd, preferred_element_type=jnp.float32)


def _make_gemm():
    return pl.pallas_call(
        _gemm_body,
        grid_spec=pltpu.PrefetchScalarGridSpec(
            num_scalar_prefetch=1,
            grid=(NBLK,),
            in_specs=[
                pl.BlockSpec((BLK, H), lambda i, bexp: (i, 0)),
                pl.BlockSpec((1, H, I), lambda i, bexp: (bexp[i], 0, 0)),
                pl.BlockSpec((1, H, I), lambda i, bexp: (bexp[i], 0, 0)),
                pl.BlockSpec((1, I, H), lambda i, bexp: (bexp[i], 0, 0)),
            ],
            out_specs=pl.BlockSpec((BLK, H), lambda i, bexp: (i, 0)),
        ),
        out_shape=jax.ShapeDtypeStruct((P_ROWS, H), jnp.float32),
        compiler_params=pltpu.CompilerParams(
            dimension_semantics=("arbitrary",),
        ),
    )


def _final_body(x_ref, wsg_ref, wsu_ref, wsd_ref, wvec_ref, g2_ref, out_ref):
    xb = x_ref[...].astype(jnp.bfloat16)
    g = jnp.dot(xb, wsg_ref[...].astype(jnp.bfloat16),
                preferred_element_type=jnp.float32)
    u = jnp.dot(xb, wsu_ref[...].astype(jnp.bfloat16),
                preferred_element_type=jnp.float32)
    a = (_silu(g) * u).astype(jnp.bfloat16)
    d = jnp.dot(a, wsd_ref[...].astype(jnp.bfloat16),
                preferred_element_type=jnp.float32)
    wv = wvec_ref[...]
    g2 = g2_ref[...]
    out_ref[...] = (d + wv[:, 0:1] * g2[:, :H] + wv[:, 1:2] * g2[:, H:])


def _make_final(tb):
    return pl.pallas_call(
        _final_body,
        grid=(T // tb,),
        out_shape=jax.ShapeDtypeStruct((T, H), jnp.float32),
        in_specs=[
            pl.BlockSpec((tb, H), lambda i: (i, 0)),
            pl.BlockSpec((H, SI), lambda i: (0, 0)),
            pl.BlockSpec((H, SI), lambda i: (0, 0)),
            pl.BlockSpec((SI, H), lambda i: (0, 0)),
            pl.BlockSpec((tb, KTOP), lambda i: (i, 0)),
            pl.BlockSpec((tb, 2 * H), lambda i: (i, 0)),
        ],
        out_specs=pl.BlockSpec((tb, H), lambda i: (i, 0)),
        compiler_params=pltpu.CompilerParams(
            dimension_semantics=("arbitrary",),
        ),
    )


def kernel(hidden_states, gate_weight, e_score_correction_bias,
           w_gate, w_up, w_down, ws_gate, ws_up, ws_down):
    x = hidden_states
    logits = jnp.dot(x, gate_weight.T).astype(jnp.float32)
    bias2 = e_score_correction_bias.reshape(1, E)

    dest, wvec, bexp = _make_router()(logits, bias2)
    dest_flat = dest.reshape(NA)

    xs = _sc_dispatch(x, dest_flat)
    d = _make_gemm()(bexp.reshape(NBLK), xs, w_gate, w_up, w_down)
    g = _sc_combine(d, dest_flat)
    g2 = g.reshape(T, 2 * H)

    out = _make_final(512)(x, ws_gate, ws_up, ws_down, wvec, g2)
    return out


# speedup vs baseline: 1.0365x; 1.0365x over previous
import functools
import jax
import jax.numpy as jnp
from jax import lax
from jax.experimental import pallas as pl
from jax.experimental.pallas import tpu as pltpu
from jax.experimental.pallas import tpu_sc as plsc

E = 8
NGROUP = 4
GSZ = E // NGROUP
SCALE = 2.5
H = 1024
I = 512
SI = 1024
T = 2048
KTOP = 2

BLK = 256
P_ROWS = T * KTOP + E * BLK
NBLK = P_ROWS // BLK
NW = 32
TOK_PER_W = T // (NW // KTOP)
CHUNK = 32


def _silu(v):
    return v / (1.0 + jnp.exp(-v))


def _top1_mask(vals, width):
    m = jnp.max(vals, axis=1, keepdims=True)
    io = lax.broadcasted_iota(jnp.int32, vals.shape, 1)
    idx = jnp.min(jnp.where(vals == m, io, width), axis=1, keepdims=True)
    return io == idx


def _router_body(logits_ref, bias_ref, d0_ref, d1_ref, wvec_ref, bexp_ref):
    logits = logits_ref[...]
    scores = 1.0 / (1.0 + jnp.exp(-logits))
    swb = scores + bias_ref[...]

    e_i = lax.broadcasted_iota(jnp.int32, (E, NGROUP), 0)
    g_i = lax.broadcasted_iota(jnp.int32, (E, NGROUP), 1)
    m_eg = ((e_i // GSZ) == g_i).astype(jnp.float32)
    g_j = lax.broadcasted_iota(jnp.int32, (NGROUP, E), 0)
    e_j = lax.broadcasted_iota(jnp.int32, (NGROUP, E), 1)
    m_ge = ((e_j // GSZ) == g_j).astype(jnp.float32)

    gs = jnp.dot(swb, m_eg, preferred_element_type=jnp.float32,
                 precision=lax.Precision.HIGHEST)
    p1 = _top1_mask(gs, NGROUP)
    p2 = _top1_mask(jnp.where(p1, -1e30, gs), NGROUP)
    gmask = jnp.logical_or(p1, p2).astype(jnp.float32)
    emask = jnp.dot(gmask, m_ge, preferred_element_type=jnp.float32,
                    precision=lax.Precision.HIGHEST)

    masked = jnp.where(emask > 0.5, swb, -1e9)
    oh1 = _top1_mask(masked, E)
    oh2 = _top1_mask(jnp.where(oh1, -1e30, masked), E)
    w1 = jnp.sum(jnp.where(oh1, scores, 0.0), axis=1, keepdims=True)
    w2 = jnp.sum(jnp.where(oh2, scores, 0.0), axis=1, keepdims=True)
    r = SCALE / (w1 + w2 + 1e-20)
    wvec_ref[...] = jnp.concatenate([w1 * r, w2 * r], axis=1)

    ohsum = jnp.where(oh1, 1.0, 0.0) + jnp.where(oh2, 1.0, 0.0)
    c = ohsum
    sh = 1
    while sh < T:
        c = c + jnp.concatenate(
            [jnp.zeros((sh, E), jnp.float32), c[: T - sh, :]], axis=0)
        sh *= 2

    counts = c[T - 1 : T, :]
    ci = counts.astype(jnp.int32)
    pc = ((ci + BLK - 1) // BLK) * BLK
    slt_i = lax.broadcasted_iota(jnp.int32, (E, E), 0)
    slt_j = lax.broadcasted_iota(jnp.int32, (E, E), 1)
    slt = (slt_i < slt_j).astype(jnp.float32)
    off = jnp.dot(pc.astype(jnp.float32), slt,
                  preferred_element_type=jnp.float32,
                  precision=lax.Precision.HIGHEST)

    slot = off + c - 1.0
    d0 = jnp.sum(jnp.where(oh1, slot, 0.0), axis=1, keepdims=True)
    d1 = jnp.sum(jnp.where(oh2, slot, 0.0), axis=1, keepdims=True)
    d0_ref[...] = d0.astype(jnp.int32)
    d1_ref[...] = d1.astype(jnp.int32)

    b_i = lax.broadcasted_iota(jnp.int32, (NBLK, E), 0)
    off_b = jnp.broadcast_to(off.astype(jnp.int32), (NBLK, E))
    cmp = (b_i * BLK >= off_b).astype(jnp.int32)
    bexp_ref[...] = jnp.sum(cmp, axis=1, keepdims=True) - 1


def _make_router():
    return pl.pallas_call(
        _router_body,
        out_shape=(
            jax.ShapeDtypeStruct((T, 1), jnp.int32),
            jax.ShapeDtypeStruct((T, 1), jnp.int32),
            jax.ShapeDtypeStruct((T, KTOP), jnp.float32),
            jax.ShapeDtypeStruct((NBLK, 1), jnp.int32),
        ),
        in_specs=[
            pl.BlockSpec((T, E), lambda: (0, 0)),
            pl.BlockSpec((1, E), lambda: (0, 0)),
        ],
        out_specs=(
            pl.BlockSpec((T, 1), lambda: (0, 0)),
            pl.BlockSpec((T, 1), lambda: (0, 0)),
            pl.BlockSpec((T, KTOP), lambda: (0, 0)),
            pl.BlockSpec((NBLK, 1), lambda: (0, 0)),
        ),
    )


_SC_MESH = plsc.VectorSubcoreMesh(core_axis_name="c", subcore_axis_name="s")


@functools.partial(
    pl.kernel,
    mesh=_SC_MESH,
    out_type=jax.ShapeDtypeStruct((P_ROWS, H), jnp.float32),
    scratch_types=[
        pltpu.VMEM((CHUNK,), jnp.int32),
        pltpu.VMEM((CHUNK, H), jnp.float32),
        pltpu.SemaphoreType.DMA,
    ],
)
def _sc_dispatch(x_hbm, d0_hbm, d1_hbm, xs_hbm, dst_v, rows_v, sem):
    nc = 2
    wid = lax.axis_index("s") * nc + lax.axis_index("c")
    k = wid // (NW // KTOP)
    t0 = (wid % (NW // KTOP)) * TOK_PER_W
    for s in range(TOK_PER_W // CHUNK):
        tb = t0 + s * CHUNK
        pltpu.sync_copy(x_hbm.at[pl.ds(tb, CHUNK)], rows_v)

        @pl.when(k == 0)
        def _():
            pltpu.sync_copy(d0_hbm.at[pl.ds(tb, CHUNK)], dst_v)

        @pl.when(k == 1)
        def _():
            pltpu.sync_copy(d1_hbm.at[pl.ds(tb, CHUNK)], dst_v)

        pltpu.async_copy(rows_v, xs_hbm.at[dst_v], sem).wait()


@functools.partial(
    pl.kernel,
    mesh=_SC_MESH,
    out_type=(
        jax.ShapeDtypeStruct((T, H), jnp.float32),
        jax.ShapeDtypeStruct((T, H), jnp.float32),
    ),
    scratch_types=[
        pltpu.VMEM((CHUNK,), jnp.int32),
        pltpu.VMEM((CHUNK, H), jnp.float32),
        pltpu.SemaphoreType.DMA,
    ],
)
def _sc_combine(d_hbm, d0_hbm, d1_hbm, g0_hbm, g1_hbm, dst_v, rows_v, sem):
    nc = 2
    wid = lax.axis_index("s") * nc + lax.axis_index("c")
    k = wid // (NW // KTOP)
    t0 = (wid % (NW // KTOP)) * TOK_PER_W
    for s in range(TOK_PER_W // CHUNK):
        tb = t0 + s * CHUNK

        @pl.when(k == 0)
        def _():
            pltpu.sync_copy(d0_hbm.at[pl.ds(tb, CHUNK)], dst_v)
            pltpu.async_copy(d_hbm.at[dst_v], rows_v, sem).wait()
            pltpu.sync_copy(rows_v, g0_hbm.at[pl.ds(tb, CHUNK)])

        @pl.when(k == 1)
        def _():
            pltpu.sync_copy(d1_hbm.at[pl.ds(tb, CHUNK)], dst_v)
            pltpu.async_copy(d_hbm.at[dst_v], rows_v, sem).wait()
            pltpu.sync_copy(rows_v, g1_hbm.at[pl.ds(tb, CHUNK)])


def _gemm_body(bexp_ref, xs_ref, wg_ref, wu_ref, wd_ref, o_ref):
    xb = xs_ref[...].astype(jnp.bfloat16)
    wg = wg_ref[0].astype(jnp.bfloat16)
    wu = wu_ref[0].astype(jnp.bfloat16)
    wd = wd_ref[0].astype(jnp.bfloat16)
    g = jnp.dot(xb, wg, preferred_element_type=jnp.float32)
    u = jnp.dot(xb, wu, preferred_element_type=jnp.float32)
    a = (_silu(g) * u).astype(jnp.bfloat16)
    o_ref[...] = jnp.dot(a, wd, preferred_element_type=jnp.float32)


def _make_gemm():
    return pl.pallas_call(
        _gemm_body,
        grid_spec=pltpu.PrefetchScalarGridSpec(
            num_scalar_prefetch=1,
            grid=(NBLK,),
            in_specs=[
                pl.BlockSpec((BLK, H), lambda i, bexp: (i, 0)),
                pl.BlockSpec((1, H, I), lambda i, bexp: (bexp[i], 0, 0)),
                pl.BlockSpec((1, H, I), lambda i, bexp: (bexp[i], 0, 0)),
                pl.BlockSpec((1, I, H), lambda i, bexp: (bexp[i], 0, 0)),
            ],
            out_specs=pl.BlockSpec((BLK, H), lambda i, bexp: (i, 0)),
        ),
        out_shape=jax.ShapeDtypeStruct((P_ROWS, H), jnp.float32),
        compiler_params=pltpu.CompilerParams(
            dimension_semantics=("arbitrary",),
        ),
    )


def _shared_body(x_ref, wsg_ref, wsu_ref, wsd_ref, dummy_ref, out_ref):
    xb = x_ref[...].astype(jnp.bfloat16)
    g = jnp.dot(xb, wsg_ref[...].astype(jnp.bfloat16),
                preferred_element_type=jnp.float32)
    u = jnp.dot(xb, wsu_ref[...].astype(jnp.bfloat16),
                preferred_element_type=jnp.float32)
    a = (_silu(g) * u).astype(jnp.bfloat16)
    out_ref[...] = jnp.dot(a, wsd_ref[...].astype(jnp.bfloat16),
                           preferred_element_type=jnp.float32)


def _make_shared(tb, dummy_spec):
    return pl.pallas_call(
        _shared_body,
        grid=(T // 2 // tb,),
        out_shape=jax.ShapeDtypeStruct((T // 2, H), jnp.float32),
        in_specs=[
            pl.BlockSpec((tb, H), lambda i: (i, 0)),
            pl.BlockSpec((H, SI), lambda i: (0, 0)),
            pl.BlockSpec((H, SI), lambda i: (0, 0)),
            pl.BlockSpec((SI, H), lambda i: (0, 0)),
            dummy_spec,
        ],
        out_specs=pl.BlockSpec((tb, H), lambda i: (i, 0)),
        compiler_params=pltpu.CompilerParams(
            dimension_semantics=("arbitrary",),
        ),
    )


def _add_body(sha_ref, shb_ref, wvec_ref, g0_ref, g1_ref, out_ref):
    i = pl.program_id(0)
    wv = wvec_ref[...]
    sh = jnp.where(i < 2, sha_ref[...], shb_ref[...])
    out_ref[...] = sh + wv[:, 0:1] * g0_ref[...] + wv[:, 1:2] * g1_ref[...]


def _make_add(tb):
    return pl.pallas_call(
        _add_body,
        grid=(T // tb,),
        out_shape=jax.ShapeDtypeStruct((T, H), jnp.float32),
        in_specs=[
            pl.BlockSpec((tb, H), lambda i: (jnp.clip(i, 0, 1), 0)),
            pl.BlockSpec((tb, H), lambda i: (jnp.clip(i - 2, 0, 1), 0)),
            pl.BlockSpec((tb, KTOP), lambda i: (i, 0)),
            pl.BlockSpec((tb, H), lambda i: (i, 0)),
            pl.BlockSpec((tb, H), lambda i: (i, 0)),
        ],
        out_specs=pl.BlockSpec((tb, H), lambda i: (i, 0)),
        compiler_params=pltpu.CompilerParams(
            dimension_semantics=("arbitrary",),
        ),
    )


def kernel(hidden_states, gate_weight, e_score_correction_bias,
           w_gate, w_up, w_down, ws_gate, ws_up, ws_down):
    x = hidden_states
    logits = jnp.dot(x, gate_weight.T).astype(jnp.float32)
    bias2 = e_score_correction_bias.reshape(1, E)

    d0, d1, wvec, bexp = _make_router()(logits, bias2)
    d0f = d0.reshape(T)
    d1f = d1.reshape(T)

    xs = _sc_dispatch(x, d0f, d1f)
    sha = _make_shared(512, pl.BlockSpec((NBLK, 1), lambda i: (0, 0)))(
        x[: T // 2], ws_gate, ws_up, ws_down, bexp)
    d = _make_gemm()(bexp.reshape(NBLK), xs, w_gate, w_up, w_down)
    g0, g1 = _sc_combine(d, d0f, d1f)
    shb = _make_shared(512, pl.BlockSpec((8, 128), lambda i: (0, 0)))(
        x[T // 2 :], ws_gate, ws_up, ws_down, d)

    out = _make_add(512)(sha, shb, wvec, g0, g1)
    return out


# speedup vs baseline: 1.1106x; 1.0715x over previous
import functools
import jax
import jax.numpy as jnp
from jax import lax
from jax.experimental import pallas as pl
from jax.experimental.pallas import tpu as pltpu
from jax.experimental.pallas import tpu_sc as plsc

E = 8
NGROUP = 4
GSZ = E // NGROUP
SCALE = 2.5
H = 1024
I = 512
SI = 1024
T = 2048
KTOP = 2

BLK = 256
P_ROWS = T * KTOP + E * BLK
NBLK = P_ROWS // BLK
NW = 32
TOK_PER_W = T // (NW // KTOP)
CHUNK = 32


def _silu(v):
    return v / (1.0 + jnp.exp(-v))


def _top1_mask(vals, width):
    m = jnp.max(vals, axis=1, keepdims=True)
    io = lax.broadcasted_iota(jnp.int32, vals.shape, 1)
    idx = jnp.min(jnp.where(vals == m, io, width), axis=1, keepdims=True)
    return io == idx


def _router_body(logits_ref, bias_ref, d0_ref, d1_ref, wvec_ref, bexp_ref):
    logits = logits_ref[...]
    scores = 1.0 / (1.0 + jnp.exp(-logits))
    swb = scores + bias_ref[...]

    e_i = lax.broadcasted_iota(jnp.int32, (E, NGROUP), 0)
    g_i = lax.broadcasted_iota(jnp.int32, (E, NGROUP), 1)
    m_eg = ((e_i // GSZ) == g_i).astype(jnp.float32)
    g_j = lax.broadcasted_iota(jnp.int32, (NGROUP, E), 0)
    e_j = lax.broadcasted_iota(jnp.int32, (NGROUP, E), 1)
    m_ge = ((e_j // GSZ) == g_j).astype(jnp.float32)

    gs = jnp.dot(swb, m_eg, preferred_element_type=jnp.float32,
                 precision=lax.Precision.HIGHEST)
    p1 = _top1_mask(gs, NGROUP)
    p2 = _top1_mask(jnp.where(p1, -1e30, gs), NGROUP)
    gmask = jnp.logical_or(p1, p2).astype(jnp.float32)
    emask = jnp.dot(gmask, m_ge, preferred_element_type=jnp.float32,
                    precision=lax.Precision.HIGHEST)

    masked = jnp.where(emask > 0.5, swb, -1e9)
    oh1 = _top1_mask(masked, E)
    oh2 = _top1_mask(jnp.where(oh1, -1e30, masked), E)
    w1 = jnp.sum(jnp.where(oh1, scores, 0.0), axis=1, keepdims=True)
    w2 = jnp.sum(jnp.where(oh2, scores, 0.0), axis=1, keepdims=True)
    r = SCALE / (w1 + w2 + 1e-20)
    wvec_ref[...] = jnp.concatenate([w1 * r, w2 * r], axis=1)

    ohsum = jnp.where(oh1, 1.0, 0.0) + jnp.where(oh2, 1.0, 0.0)
    c = ohsum
    sh = 1
    while sh < T:
        c = c + jnp.concatenate(
            [jnp.zeros((sh, E), jnp.float32), c[: T - sh, :]], axis=0)
        sh *= 2

    counts = c[T - 1 : T, :]
    ci = counts.astype(jnp.int32)
    pc = ((ci + BLK - 1) // BLK) * BLK
    slt_i = lax.broadcasted_iota(jnp.int32, (E, E), 0)
    slt_j = lax.broadcasted_iota(jnp.int32, (E, E), 1)
    slt = (slt_i < slt_j).astype(jnp.float32)
    off = jnp.dot(pc.astype(jnp.float32), slt,
                  preferred_element_type=jnp.float32,
                  precision=lax.Precision.HIGHEST)

    slot = off + c - 1.0
    d0 = jnp.sum(jnp.where(oh1, slot, 0.0), axis=1, keepdims=True)
    d1 = jnp.sum(jnp.where(oh2, slot, 0.0), axis=1, keepdims=True)
    d0_ref[...] = d0.astype(jnp.int32)
    d1_ref[...] = d1.astype(jnp.int32)

    b_i = lax.broadcasted_iota(jnp.int32, (NBLK, E), 0)
    off_b = jnp.broadcast_to(off.astype(jnp.int32), (NBLK, E))
    cmp = (b_i * BLK >= off_b).astype(jnp.int32)
    bexp_ref[...] = jnp.sum(cmp, axis=1, keepdims=True) - 1


def _make_router():
    return pl.pallas_call(
        _router_body,
        out_shape=(
            jax.ShapeDtypeStruct((T, 1), jnp.int32),
            jax.ShapeDtypeStruct((T, 1), jnp.int32),
            jax.ShapeDtypeStruct((T, KTOP), jnp.float32),
            jax.ShapeDtypeStruct((NBLK, 1), jnp.int32),
        ),
        in_specs=[
            pl.BlockSpec((T, E), lambda: (0, 0)),
            pl.BlockSpec((1, E), lambda: (0, 0)),
        ],
        out_specs=(
            pl.BlockSpec((T, 1), lambda: (0, 0)),
            pl.BlockSpec((T, 1), lambda: (0, 0)),
            pl.BlockSpec((T, KTOP), lambda: (0, 0)),
            pl.BlockSpec((NBLK, 1), lambda: (0, 0)),
        ),
    )


_SC_MESH = plsc.VectorSubcoreMesh(core_axis_name="c", subcore_axis_name="s")


@functools.partial(
    pl.kernel,
    mesh=_SC_MESH,
    out_type=jax.ShapeDtypeStruct((P_ROWS, H), jnp.float32),
    scratch_types=[
        pltpu.VMEM((CHUNK,), jnp.int32),
        pltpu.VMEM((CHUNK, H), jnp.float32),
        pltpu.SemaphoreType.DMA,
    ],
)
def _sc_dispatch(x_hbm, d0_hbm, d1_hbm, xs_hbm, dst_v, rows_v, sem):
    nc = 2
    wid = lax.axis_index("s") * nc + lax.axis_index("c")
    k = wid // (NW // KTOP)
    t0 = (wid % (NW // KTOP)) * TOK_PER_W
    for s in range(TOK_PER_W // CHUNK):
        tb = t0 + s * CHUNK
        pltpu.sync_copy(x_hbm.at[pl.ds(tb, CHUNK)], rows_v)

        @pl.when(k == 0)
        def _():
            pltpu.sync_copy(d0_hbm.at[pl.ds(tb, CHUNK)], dst_v)

        @pl.when(k == 1)
        def _():
            pltpu.sync_copy(d1_hbm.at[pl.ds(tb, CHUNK)], dst_v)

        pltpu.async_copy(rows_v, xs_hbm.at[dst_v], sem).wait()


@functools.partial(
    pl.kernel,
    mesh=_SC_MESH,
    out_type=(
        jax.ShapeDtypeStruct((T, H), jnp.float32),
        jax.ShapeDtypeStruct((T, H), jnp.float32),
    ),
    scratch_types=[
        pltpu.VMEM((CHUNK,), jnp.int32),
        pltpu.VMEM((CHUNK, H), jnp.float32),
        pltpu.SemaphoreType.DMA,
    ],
)
def _sc_combine(d_hbm, d0_hbm, d1_hbm, g0_hbm, g1_hbm, dst_v, rows_v, sem):
    nc = 2
    wid = lax.axis_index("s") * nc + lax.axis_index("c")
    k = wid // (NW // KTOP)
    t0 = (wid % (NW // KTOP)) * TOK_PER_W
    for s in range(TOK_PER_W // CHUNK):
        tb = t0 + s * CHUNK

        @pl.when(k == 0)
        def _():
            pltpu.sync_copy(d0_hbm.at[pl.ds(tb, CHUNK)], dst_v)
            pltpu.async_copy(d_hbm.at[dst_v], rows_v, sem).wait()
            pltpu.sync_copy(rows_v, g0_hbm.at[pl.ds(tb, CHUNK)])

        @pl.when(k == 1)
        def _():
            pltpu.sync_copy(d1_hbm.at[pl.ds(tb, CHUNK)], dst_v)
            pltpu.async_copy(d_hbm.at[dst_v], rows_v, sem).wait()
            pltpu.sync_copy(rows_v, g1_hbm.at[pl.ds(tb, CHUNK)])


def _gemm_body(bexp_ref, xs_ref, wg_ref, wu_ref, wd_ref, o_ref):
    xb = xs_ref[...].astype(jnp.bfloat16)
    wg = wg_ref[0].astype(jnp.bfloat16)
    wu = wu_ref[0].astype(jnp.bfloat16)
    wd = wd_ref[0].astype(jnp.bfloat16)
    g = jnp.dot(xb, wg, preferred_element_type=jnp.float32)
    u = jnp.dot(xb, wu, preferred_element_type=jnp.float32)
    a = (_silu(g) * u).astype(jnp.bfloat16)
    o_ref[...] = jnp.dot(a, wd, preferred_element_type=jnp.float32)


def _make_gemm():
    return pl.pallas_call(
        _gemm_body,
        grid_spec=pltpu.PrefetchScalarGridSpec(
            num_scalar_prefetch=1,
            grid=(NBLK,),
            in_specs=[
                pl.BlockSpec((BLK, H), lambda i, bexp: (i, 0)),
                pl.BlockSpec((1, H, I), lambda i, bexp: (bexp[i], 0, 0)),
                pl.BlockSpec((1, H, I), lambda i, bexp: (bexp[i], 0, 0)),
                pl.BlockSpec((1, I, H), lambda i, bexp: (bexp[i], 0, 0)),
            ],
            out_specs=pl.BlockSpec((BLK, H), lambda i, bexp: (i, 0)),
        ),
        out_shape=jax.ShapeDtypeStruct((P_ROWS, H), jnp.float32),
        compiler_params=pltpu.CompilerParams(
            dimension_semantics=("arbitrary",),
        ),
    )


def _shared_body(x_ref, wsg_ref, wsu_ref, wsd_ref, dummy_ref, out_ref):
    xb = x_ref[...].astype(jnp.bfloat16)
    g = jnp.dot(xb, wsg_ref[...].astype(jnp.bfloat16),
                preferred_element_type=jnp.float32)
    u = jnp.dot(xb, wsu_ref[...].astype(jnp.bfloat16),
                preferred_element_type=jnp.float32)
    a = (_silu(g) * u).astype(jnp.bfloat16)
    out_ref[...] = jnp.dot(a, wsd_ref[...].astype(jnp.bfloat16),
                           preferred_element_type=jnp.float32)


def _make_shared(tb, off, dummy_spec):
    return pl.pallas_call(
        _shared_body,
        grid=(T // 2 // tb,),
        out_shape=jax.ShapeDtypeStruct((T // 2, H), jnp.float32),
        in_specs=[
            pl.BlockSpec((tb, H), lambda i: (i + off, 0)),
            pl.BlockSpec((H, SI), lambda i: (0, 0)),
            pl.BlockSpec((H, SI), lambda i: (0, 0)),
            pl.BlockSpec((SI, H), lambda i: (0, 0)),
            dummy_spec,
        ],
        out_specs=pl.BlockSpec((tb, H), lambda i: (i, 0)),
        compiler_params=pltpu.CompilerParams(
            dimension_semantics=("arbitrary",),
        ),
    )


def _add_body(sha_ref, shb_ref, wvec_ref, g0_ref, g1_ref, out_ref):
    i = pl.program_id(0)
    wv = wvec_ref[...]
    sh = jnp.where(i < 2, sha_ref[...], shb_ref[...])
    out_ref[...] = sh + wv[:, 0:1] * g0_ref[...] + wv[:, 1:2] * g1_ref[...]


def _make_add(tb):
    return pl.pallas_call(
        _add_body,
        grid=(T // tb,),
        out_shape=jax.ShapeDtypeStruct((T, H), jnp.float32),
        in_specs=[
            pl.BlockSpec((tb, H), lambda i: (jnp.clip(i, 0, 1), 0)),
            pl.BlockSpec((tb, H), lambda i: (jnp.clip(i - 2, 0, 1), 0)),
            pl.BlockSpec((tb, KTOP), lambda i: (i, 0)),
            pl.BlockSpec((tb, H), lambda i: (i, 0)),
            pl.BlockSpec((tb, H), lambda i: (i, 0)),
        ],
        out_specs=pl.BlockSpec((tb, H), lambda i: (i, 0)),
        compiler_params=pltpu.CompilerParams(
            dimension_semantics=("arbitrary",),
        ),
    )


def kernel(hidden_states, gate_weight, e_score_correction_bias,
           w_gate, w_up, w_down, ws_gate, ws_up, ws_down):
    x = hidden_states
    logits = jnp.dot(x, gate_weight.T).astype(jnp.float32)
    bias2 = e_score_correction_bias.reshape(1, E)

    d0, d1, wvec, bexp = _make_router()(logits, bias2)
    d0f = d0.reshape(T)
    d1f = d1.reshape(T)

    sha = _make_shared(512, 0, pl.BlockSpec((NBLK, 1), lambda i: (0, 0)))(
        x, ws_gate, ws_up, ws_down, bexp)
    xs = _sc_dispatch(x, d0f, d1f)
    d = _make_gemm()(bexp.reshape(NBLK), xs, w_gate, w_up, w_down)
    g0, g1 = _sc_combine(d, d0f, d1f)
    shb = _make_shared(512, 2, pl.BlockSpec((8, 128), lambda i: (0, 0)))(
        x, ws_gate, ws_up, ws_down, d)

    out = _make_add(512)(sha, shb, wvec, g0, g1)
    return out
